# baseline (speedup 1.0000x reference)
import jax
import jax.numpy as jnp
from jax.experimental import pallas as pl
from jax.experimental.pallas import tpu as pltpu

SEQ = 2048
EMB = 32
VOCAB = 129
TW = 2 * SEQ
NB = 128


def _tc_body(span_ref, tablet_ref, out_ref):
    g = pl.program_id(0)
    s0 = (NB - 1) - g
    span = span_ref[0]

    vv = jax.lax.broadcasted_iota(jnp.int32, (VOCAB, TW), 0)
    kk = jax.lax.broadcasted_iota(jnp.int32, (VOCAB, TW), 1) + (s0 - (SEQ - 1))
    idx = jnp.clip(jnp.abs(kk), 0, span)
    oh = (vv == idx).astype(jnp.float32)
    t_s0 = jnp.dot(tablet_ref[...], oh, preferred_element_type=jnp.float32,
                   precision=jax.lax.Precision.HIGHEST)

    for a in range(SEQ // NB):
        off = NB * (SEQ // NB - 1 - a)
        out_ref[a, 0] = t_s0[:, off:off + SEQ]


def kernel(table, seq_len, layer_attention_span):
    span = jnp.asarray(layer_attention_span, jnp.int32).reshape(1)
    tablet = table.T

    out4 = pl.pallas_call(
        _tc_body,
        grid=(NB,),
        in_specs=[
            pl.BlockSpec(memory_space=pltpu.SMEM),
            pl.BlockSpec((EMB, VOCAB), lambda g: (0, 0)),
        ],
        out_specs=pl.BlockSpec((SEQ // NB, 1, EMB, SEQ), lambda g: (0, g, 0, 0)),
        out_shape=jax.ShapeDtypeStruct((SEQ // NB, NB, EMB, SEQ), jnp.float32),
        compiler_params=pltpu.CompilerParams(
            dimension_semantics=("arbitrary",),
        ),
    )(span, tablet)

    out_phys = out4.reshape(SEQ, EMB, SEQ)
    return jnp.transpose(out_phys, (0, 2, 1))


# speedup vs baseline: 1.3788x; 1.3788x over previous
import jax
import jax.numpy as jnp
from jax.experimental import pallas as pl
from jax.experimental.pallas import tpu as pltpu

SEQ = 2048
EMB = 32
VOCAB = 129
TW = 2 * SEQ
NB = 128


WIN0 = SEQ - 2 * NB
WINW = 3 * NB


def _tc_body(span_ref, tablet_ref, out_ref):
    g = pl.program_id(0)
    s0 = (NB - 1) - g
    span = span_ref[0]

    vv = jax.lax.broadcasted_iota(jnp.int32, (VOCAB, WINW), 0)
    kk = jax.lax.broadcasted_iota(jnp.int32, (VOCAB, WINW), 1) + (
        WIN0 + s0 - (SEQ - 1))
    idx = jnp.clip(jnp.abs(kk), 0, span)
    oh = (vv == idx).astype(jnp.float32)
    t_win = jnp.dot(tablet_ref[...], oh, preferred_element_type=jnp.float32,
                    precision=jax.lax.Precision.HIGHEST)
    filler = t_win[:, :1]
    t_s0 = jnp.concatenate(
        [jnp.broadcast_to(filler, (EMB, WIN0)),
         t_win,
         jnp.broadcast_to(filler, (EMB, TW - WIN0 - WINW))],
        axis=1,
    )

    for a in range(SEQ // NB):
        off = NB * (SEQ // NB - 1 - a)
        out_ref[a, 0] = t_s0[:, off:off + SEQ]


def kernel(table, seq_len, layer_attention_span):
    span = jnp.asarray(layer_attention_span, jnp.int32).reshape(1)
    tablet = table.T

    out4 = pl.pallas_call(
        _tc_body,
        grid=(NB,),
        in_specs=[
            pl.BlockSpec(memory_space=pltpu.SMEM),
            pl.BlockSpec((EMB, VOCAB), lambda g: (0, 0)),
        ],
        out_specs=pl.BlockSpec((SEQ // NB, 1, EMB, SEQ), lambda g: (0, g, 0, 0)),
        out_shape=jax.ShapeDtypeStruct((SEQ // NB, NB, EMB, SEQ), jnp.float32),
        compiler_params=pltpu.CompilerParams(
            dimension_semantics=("arbitrary",),
        ),
    )(span, tablet)

    out_phys = out4.reshape(SEQ, EMB, SEQ)
    return jnp.transpose(out_phys, (0, 2, 1))
